# baseline (speedup 1.0000x reference)
import functools
import math

import jax
import jax.numpy as jnp
from jax import lax
from jax.experimental import pallas as pl
from jax.experimental.pallas import tpu as pltpu
from jax.experimental.pallas import tpu_sc as plsc

D_MODEL = 64
D_PAD = 128
SCALE = math.sqrt(D_MODEL)
NUM_WORKERS = 32


def _sc_embed(idx_flat, table_pad, batch, seq):
    b_per_w = batch // NUM_WORKERS
    i_per_w = b_per_w * seq
    mesh = plsc.VectorSubcoreMesh(core_axis_name="c", subcore_axis_name="s")

    @functools.partial(
        pl.kernel,
        out_type=jax.ShapeDtypeStruct((batch, seq, D_MODEL), jnp.float32),
        mesh=mesh,
        scratch_types=[
            pltpu.VMEM((i_per_w,), jnp.int32),
            pltpu.VMEM((seq, D_PAD), jnp.float32),
            pltpu.VMEM((seq, D_PAD), jnp.float32),
            pltpu.VMEM((seq, D_MODEL), jnp.float32),
            pltpu.VMEM((seq, D_MODEL), jnp.float32),
            pltpu.SemaphoreType.DMA,
            pltpu.SemaphoreType.DMA,
            pltpu.SemaphoreType.DMA,
            pltpu.SemaphoreType.DMA,
        ],
        compiler_params=pltpu.CompilerParams(use_tc_tiling_on_sc=True),
    )
    def k(idx_hbm, table_hbm, out_hbm, idx_slab, ga0, ga1, sb0, sb1,
          gsem0, gsem1, ssem0, ssem1):
        ga = (ga0, ga1)
        sb = (sb0, sb1)
        gsem = (gsem0, gsem1)
        ssem = (ssem0, ssem1)
        wid = lax.axis_index("s") * 2 + lax.axis_index("c")
        base = wid * b_per_w

        pltpu.sync_copy(idx_hbm.at[pl.ds(base * seq, i_per_w)], idx_slab)

        def start_gather(b, buf):
            pltpu.async_copy(
                table_hbm.at[idx_slab.at[pl.ds(b * seq, seq)]], ga[buf],
                gsem[buf])

        def scale_buf(buf):
            def scale_row(i, carry2):
                for j in range(D_MODEL // 16):
                    s = pl.ds(j * 16, 16)
                    sb[buf][i, s] = ga[buf][i, s] * SCALE
                return carry2
            lax.fori_loop(0, seq, scale_row, 0, unroll=4)

        def start_store(b, buf):
            pltpu.async_copy(sb[buf], out_hbm.at[base + b], ssem[buf])

        def wait_gather(buf):
            pltpu.make_async_copy(
                table_hbm.at[idx_slab.at[pl.ds(0, seq)]], ga[buf],
                gsem[buf]).wait()

        def wait_store(buf):
            pltpu.make_async_copy(sb[buf], out_hbm.at[0], ssem[buf]).wait()

        start_gather(0, 0)

        def outer(g, carry):
            for buf in (0, 1):
                b = 2 * g + buf
                other = 1 - buf
                wait_gather(buf)
                if buf == 1:
                    wait_store(other)
                else:
                    @pl.when(g > 0)
                    def _():
                        wait_store(other)
                if buf == 0:
                    start_gather(b + 1, other)
                else:
                    @pl.when(2 * g + 2 < b_per_w)
                    def _():
                        start_gather(b + 1, other)
                scale_buf(buf)
                start_store(b, buf)
            return carry

        lax.fori_loop(0, b_per_w // 2, outer, 0)
        wait_store(1)

    return k(idx_flat, table_pad)


def kernel(x, table):
    batch, seq = x.shape
    idx_flat = x.reshape(batch * seq)
    table_pad = jnp.pad(table, ((0, 0), (0, D_PAD - D_MODEL)))
    return _sc_embed(idx_flat, table_pad, batch, seq)


# speedup vs baseline: 1.1894x; 1.1894x over previous
import functools
import math

import jax
import jax.numpy as jnp
from jax import lax
from jax.experimental import pallas as pl
from jax.experimental.pallas import tpu as pltpu
from jax.experimental.pallas import tpu_sc as plsc

D_MODEL = 64
D_PAD = 128
SCALE = math.sqrt(D_MODEL)
NUM_WORKERS = 32
CHUNK = 80
NBUF = 4


def _sc_gather(idx_flat, table_pad, n_idx):
    i_per_w = n_idx // NUM_WORKERS
    n_chunks = i_per_w // CHUNK
    assert n_chunks % NBUF == 0 and CHUNK % 8 == 0
    mesh = plsc.VectorSubcoreMesh(core_axis_name="c", subcore_axis_name="s")

    @functools.partial(
        pl.kernel,
        out_type=jax.ShapeDtypeStruct((n_idx, D_MODEL), jnp.float32),
        mesh=mesh,
        scratch_types=[
            pltpu.VMEM((i_per_w,), jnp.int32),
            pltpu.VMEM((CHUNK, D_PAD), jnp.float32),
            pltpu.VMEM((CHUNK, D_PAD), jnp.float32),
            pltpu.VMEM((CHUNK, D_PAD), jnp.float32),
            pltpu.VMEM((CHUNK, D_PAD), jnp.float32),
            pltpu.VMEM((CHUNK, D_MODEL), jnp.float32),
            pltpu.VMEM((CHUNK, D_MODEL), jnp.float32),
            pltpu.SemaphoreType.DMA,
            pltpu.SemaphoreType.DMA,
            pltpu.SemaphoreType.DMA,
            pltpu.SemaphoreType.DMA,
            pltpu.SemaphoreType.DMA,
            pltpu.SemaphoreType.DMA,
        ],
        compiler_params=pltpu.CompilerParams(use_tc_tiling_on_sc=True),
    )
    def k(idx_hbm, table_hbm, out_hbm, idx_slab, g0, g1, g2, g3, sb0, sb1,
          gs0, gs1, gs2, gs3, ss0, ss1):
        ga = (g0, g1, g2, g3)
        sb = (sb0, sb1)
        gsem = (gs0, gs1, gs2, gs3)
        ssem = (ss0, ss1)
        wid = lax.axis_index("s") * 2 + lax.axis_index("c")
        base = wid * i_per_w

        pltpu.sync_copy(idx_hbm.at[pl.ds(base, i_per_w)], idx_slab)

        def start_gather(ci, buf):
            pltpu.async_copy(
                table_hbm.at[idx_slab.at[pl.ds(ci * CHUNK, CHUNK)]], ga[buf],
                gsem[buf])

        def wait_gather(buf):
            pltpu.make_async_copy(
                table_hbm.at[idx_slab.at[pl.ds(0, CHUNK)]], ga[buf],
                gsem[buf]).wait()

        def scale_buf(gbuf, cbuf):
            def scale_row(i, carry):
                for j in range(D_MODEL // 16):
                    s = pl.ds(j * 16, 16)
                    sb[cbuf][i, s] = ga[gbuf][i, s] * SCALE
                return carry
            lax.fori_loop(0, CHUNK, scale_row, 0, unroll=4)

        def start_store(ci, cbuf):
            pltpu.async_copy(sb[cbuf],
                             out_hbm.at[pl.ds(base + ci * CHUNK, CHUNK)],
                             ssem[cbuf])

        def wait_store(cbuf):
            pltpu.make_async_copy(sb[cbuf], out_hbm.at[pl.ds(0, CHUNK)],
                                  ssem[cbuf]).wait()

        for p in range(NBUF - 1):
            start_gather(p, p)

        def outer(g, carry):
            for buf in range(NBUF):
                ci = NBUF * g + buf
                nxt = (buf + NBUF - 1) % NBUF
                cbuf = buf % 2
                wait_gather(buf)
                @pl.when(ci + NBUF - 1 < n_chunks)
                def _():
                    start_gather(ci + NBUF - 1, nxt)
                @pl.when(ci >= 2)
                def _():
                    wait_store(cbuf)
                scale_buf(buf, cbuf)
                start_store(ci, cbuf)
            return carry

        lax.fori_loop(0, n_chunks // NBUF, outer, 0)
        wait_store(0)
        wait_store(1)

    return k(idx_flat, table_pad)


def kernel(x, table):
    batch, seq = x.shape
    idx_flat = x.reshape(batch * seq)
    table_pad = jnp.pad(table, ((0, 0), (0, D_PAD - D_MODEL)))
    out = _sc_gather(idx_flat, table_pad, batch * seq)
    return out.reshape(batch, seq, D_MODEL)
